# baseline (speedup 1.0000x reference)
import jax
import jax.numpy as jnp
from jax import lax
from jax.experimental import pallas as pl
from jax.experimental.pallas import tpu as pltpu
from jax.experimental.pallas import tpu_sc as plsc

VOCAB = 32000
D = 768
SEQ = 200
BATCH = 1024

NC = 2
NS = 16
NW = NC * NS

TOKENS = BATCH * SEQ
CHUNK = 40
NTASK = TOKENS // CHUNK
TASKS_PER_W = NTASK // NW
LANES = 16
VECS_PER_ROW = D // LANES


def _emb_body(table, idx, pos, out, shared_pos, idx_v, pos_v, rows_v, sem):
    cid = lax.axis_index("c")
    sid = lax.axis_index("s")
    wid = sid * NC + cid

    @pl.when(sid == 0)
    def _():
        pltpu.sync_copy(pos, shared_pos)

    plsc.subcore_barrier()

    def task(k, carry):
        t = wid * TASKS_PER_W + k
        base = t * CHUNK
        l0 = lax.rem(base, SEQ)
        pltpu.sync_copy(idx.at[pl.ds(base, CHUNK)], idx_v)
        gather = pltpu.async_copy(table.at[idx_v], rows_v, sem)
        pltpu.sync_copy(shared_pos.at[pl.ds(l0, CHUNK)], pos_v)
        gather.wait()

        def add_row(r, c2):
            for c in range(VECS_PER_ROW):
                sl = pl.ds(c * LANES, LANES)
                rows_v[r, sl] = rows_v[r, sl] + pos_v[r, sl]
            return c2

        lax.fori_loop(0, CHUNK, add_row, 0)
        pltpu.sync_copy(rows_v, out.at[pl.ds(base, CHUNK)])
        return carry

    lax.fori_loop(0, TASKS_PER_W, task, 0)


@jax.jit
def kernel(token_ids, tok_embed, pos_embed):
    ids_flat = token_ids.reshape(-1).astype(jnp.int32)
    pos2d = pos_embed[0, :SEQ, :]

    mesh = plsc.VectorSubcoreMesh(core_axis_name="c", subcore_axis_name="s")
    out = pl.kernel(
        _emb_body,
        out_type=jax.ShapeDtypeStruct((TOKENS, D), jnp.float32),
        mesh=mesh,
        scratch_types=[
            pltpu.VMEM_SHARED((SEQ, D), jnp.float32),
            pltpu.VMEM((CHUNK,), jnp.int32),
            pltpu.VMEM((CHUNK, D), jnp.float32),
            pltpu.VMEM((CHUNK, D), jnp.float32),
            pltpu.SemaphoreType.DMA,
        ],
    )(tok_embed, ids_flat, pos2d)
    return out.reshape(BATCH, SEQ, D)
